# baseline (speedup 1.0000x reference)
import functools

import jax
import jax.numpy as jnp
from jax import lax
from jax.experimental import pallas as pl
from jax.experimental.pallas import tpu as pltpu
from jax.experimental.pallas import tpu_sc as plsc

NUM_CORES = 2
NUM_SUBCORES = 16
NUM_WORKERS = NUM_CORES * NUM_SUBCORES
GROUP = 128


@functools.partial(jax.jit, static_argnames=("n_groups",))
def _sc_gather(idx3, weight, n_groups):
  b_per_w = n_groups * GROUP
  total = NUM_WORKERS * b_per_w
  d = weight.shape[1]
  mesh = plsc.VectorSubcoreMesh(
      core_axis_name="c", subcore_axis_name="s",
      num_cores=NUM_CORES, num_subcores=NUM_SUBCORES)

  @functools.partial(
      pl.kernel,
      out_type=jax.ShapeDtypeStruct((total, d), jnp.float32),
      mesh=mesh,
      scratch_types=[
          pltpu.VMEM((n_groups, GROUP), jnp.int32),
          pltpu.VMEM((GROUP, d), jnp.float32),
          pltpu.SemaphoreType.DMA,
      ],
      compiler_params=pltpu.CompilerParams(use_tc_tiling_on_sc=False),
  )
  def body(idx_hbm, table_hbm, out_hbm, idx_v, rows_v, gsem):
    wid = lax.axis_index("s") * NUM_CORES + lax.axis_index("c")
    base = wid * b_per_w
    pltpu.sync_copy(idx_hbm.at[wid], idx_v)

    @pl.loop(0, n_groups)
    def _(g):
      pltpu.async_copy(table_hbm.at[idx_v.at[g]], rows_v, gsem).wait()
      pltpu.sync_copy(rows_v, out_hbm.at[pl.ds(base + g * GROUP, GROUP)])

  return body(idx3, weight)


def kernel(input_, weight):
  b0, b1 = input_.shape
  total = b0 * b1
  b_per_w = total // NUM_WORKERS
  n_groups = b_per_w // GROUP
  idx3 = input_.astype(jnp.int32).reshape(NUM_WORKERS, n_groups, GROUP)
  out = _sc_gather(idx3, weight, n_groups)
  return out.reshape(b0, b1, weight.shape[1])


# speedup vs baseline: 1.0705x; 1.0705x over previous
import functools

import jax
import jax.numpy as jnp
from jax import lax
from jax.experimental import pallas as pl
from jax.experimental.pallas import tpu as pltpu
from jax.experimental.pallas import tpu_sc as plsc

NUM_CORES = 2
NUM_SUBCORES = 16
NUM_WORKERS = NUM_CORES * NUM_SUBCORES
GROUP = 128
K = 4


@functools.partial(jax.jit, static_argnames=("n_groups",))
def _sc_gather(idx3, weight, n_groups):
  b_per_w = n_groups * GROUP
  total = NUM_WORKERS * b_per_w
  d = weight.shape[1]
  mesh = plsc.VectorSubcoreMesh(
      core_axis_name="c", subcore_axis_name="s",
      num_cores=NUM_CORES, num_subcores=NUM_SUBCORES)

  @functools.partial(
      pl.kernel,
      out_type=jax.ShapeDtypeStruct((total, d), jnp.float32),
      mesh=mesh,
      scratch_types=[
          pltpu.VMEM((n_groups, GROUP), jnp.int32),
          pltpu.VMEM((2, K * GROUP, d), jnp.float32),
          pltpu.SemaphoreType.DMA,
          pltpu.SemaphoreType.DMA,
          pltpu.SemaphoreType.DMA,
          pltpu.SemaphoreType.DMA,
      ],
      compiler_params=pltpu.CompilerParams(use_tc_tiling_on_sc=False),
  )
  def body(idx_hbm, table_hbm, out_hbm, idx_v, rows_v, ga, gb, sa, sb):
    wid = lax.axis_index("s") * NUM_CORES + lax.axis_index("c")
    base = wid * b_per_w
    n_rounds = n_groups // K
    pltpu.sync_copy(idx_hbm.at[wid], idx_v)

    def fire(r, buf, gsem):
      for k in range(K):
        pltpu.async_copy(table_hbm.at[idx_v.at[r * K + k]],
                         buf.at[pl.ds(k * GROUP, GROUP)], gsem)

    def drain(buf, gsem):
      for k in range(K):
        pltpu.make_async_copy(table_hbm.at[idx_v.at[0]],
                              buf.at[pl.ds(k * GROUP, GROUP)], gsem).wait()

    def store(r, buf, ssem):
      pltpu.async_copy(buf, out_hbm.at[pl.ds(base + r * K * GROUP, K * GROUP)],
                       ssem)

    def wait_store(buf, ssem):
      pltpu.make_async_copy(buf, out_hbm.at[pl.ds(base, K * GROUP)], ssem).wait()

    fire(0, rows_v.at[0], ga)
    fire(1, rows_v.at[1], gb)

    @pl.loop(0, n_rounds - 2, step=2)
    def _(r):
      drain(rows_v.at[0], ga)
      store(r, rows_v.at[0], sa)
      drain(rows_v.at[1], gb)
      store(r + 1, rows_v.at[1], sb)
      wait_store(rows_v.at[0], sa)
      fire(r + 2, rows_v.at[0], ga)
      wait_store(rows_v.at[1], sb)
      fire(r + 3, rows_v.at[1], gb)

    drain(rows_v.at[0], ga)
    pltpu.sync_copy(rows_v.at[0],
                    out_hbm.at[pl.ds(base + (n_rounds - 2) * K * GROUP, K * GROUP)])
    drain(rows_v.at[1], gb)
    pltpu.sync_copy(rows_v.at[1],
                    out_hbm.at[pl.ds(base + (n_rounds - 1) * K * GROUP, K * GROUP)])

  return body(idx3, weight)


def kernel(input_, weight):
  b0, b1 = input_.shape
  total = b0 * b1
  b_per_w = total // NUM_WORKERS
  n_groups = b_per_w // GROUP
  idx3 = input_.astype(jnp.int32).reshape(NUM_WORKERS, n_groups, GROUP)
  out = _sc_gather(idx3, weight, n_groups)
  return out.reshape(b0, b1, weight.shape[1])
